# baseline (speedup 1.0000x reference)
import functools

import jax
import jax.numpy as jnp
from jax import lax
from jax.experimental import pallas as pl
from jax.experimental.pallas import tpu as pltpu
from jax.experimental.pallas import tpu_sc as plsc

NC = 2
NS = 16
NW = NC * NS
BB = 128
CH = 32
L = 16


def _emb_body(seq, d, table_hbm, idx_hbm, out_hbm, idx_v, rows_v, obuf,
              gsem0, gsem1, wsem0, wsem1):
    wid = lax.axis_index("s") * NC + lax.axis_index("c")

    pltpu.sync_copy(idx_hbm.at[wid], idx_v)

    gsems = (gsem0, gsem1)
    wsems = (wsem0, wsem1)

    def issue(s, h, slot):
        def g16(g, carry):
            vec = idx_v[s, 0, pl.ds(h * CH + g * L, L)]
            tbv = (vec >> 3) * 8
            for u in range(L):
                tb = pl.multiple_of(tbv[u], 8)
                j = g * L + u
                pltpu.async_copy(
                    table_hbm.at[pl.ds(tb, 8), :],
                    rows_v.at[slot, pl.ds(j * 8, 8), :],
                    gsems[slot],
                )
            return carry

        lax.fori_loop(0, CH // L, g16, 0, unroll=True)

    def drain(slot):
        pltpu.make_async_copy(table_hbm.at[pl.ds(0, CH * 8), :],
                              rows_v.at[slot], gsems[slot]).wait()

    def transpose(s, h, slot, obs):
        rows = rows_v.at[slot]
        for g in range(CH // L):
            vec = idx_v[s, 0, pl.ds(h * CH + g * L, L)]
            rowv = (lax.iota(jnp.int32, L) + g * L) * 8 + (vec & 7)

            def per_f(f, carry):
                colv = jnp.zeros((L,), jnp.int32) + f
                vals = plsc.load_gather(rows, [rowv, colv])
                obuf[obs, f, pl.ds(h * CH + g * L, L)] = vals
                return carry

            lax.fori_loop(0, d, per_f, 0, unroll=8)

    def write(s, obs):
        pltpu.async_copy(obuf.at[obs], out_hbm.at[s, :, pl.ds(wid * BB, BB)],
                         wsems[obs])

    def wwait(obs):
        pltpu.make_async_copy(obuf.at[obs], out_hbm.at[0, :, pl.ds(0, BB)],
                              wsems[obs]).wait()

    issue(0, 0, 0)
    issue(0, 1, 1)

    def body(i, carry):
        for obs in range(2):
            s = 2 * i + obs

            @pl.when(s >= 2)
            def _():
                wwait(obs)

            for h in range(4):
                slot = h % 2
                drain(slot)
                transpose(s, h, slot, obs)
                if h < 2:
                    issue(s, h + 2, slot)
                else:
                    @pl.when(s + 1 < seq)
                    def _():
                        issue(s + 1, h - 2, slot)

            write(s, obs)
        return carry

    lax.fori_loop(0, seq // 2, body, 0, unroll=False)

    wwait(0)
    wwait(1)


def kernel(token_ids, weights):
    bsz, seq = token_ids.shape
    n, d = weights.shape
    assert bsz == NW * BB and d == 64 and seq % 2 == 0

    idx4 = (token_ids.astype(jnp.int32)
            .reshape(NW, BB, seq)
            .transpose(0, 2, 1)
            .reshape(NW, seq, 1, BB))

    emb = functools.partial(
        pl.kernel,
        mesh=plsc.VectorSubcoreMesh(core_axis_name="c", subcore_axis_name="s"),
        out_type=jax.ShapeDtypeStruct((seq, d, bsz), jnp.float32),
        scratch_types=[
            pltpu.VMEM((seq, 1, BB), jnp.int32),
            pltpu.VMEM((2, CH * 8, d), jnp.float32),
            pltpu.VMEM((2, d, BB), jnp.float32),
            pltpu.SemaphoreType.DMA,
            pltpu.SemaphoreType.DMA,
            pltpu.SemaphoreType.DMA,
            pltpu.SemaphoreType.DMA,
        ],
        compiler_params=pltpu.CompilerParams(needs_layout_passes=False),
    )(functools.partial(_emb_body, seq, d))

    out_t = emb(weights, idx4)
    return out_t.transpose(2, 0, 1)


# speedup vs baseline: 1.0534x; 1.0534x over previous
import functools

import jax
import jax.numpy as jnp
from jax import lax
from jax.experimental import pallas as pl
from jax.experimental.pallas import tpu as pltpu
from jax.experimental.pallas import tpu_sc as plsc

NC = 2
NS = 16
NW = NC * NS
BB = 128
CH = 16
L = 16
NBUF = 4
UPS = BB // CH


def _emb_body(seq, d, table_hbm, idx_hbm, out_hbm, idx_v, rows_v, obuf,
              gsem0, gsem1, gsem2, gsem3, wsem0, wsem1):
    wid = lax.axis_index("s") * NC + lax.axis_index("c")

    pltpu.sync_copy(idx_hbm.at[wid], idx_v)

    gsems = (gsem0, gsem1, gsem2, gsem3)
    wsems = (wsem0, wsem1)

    def issue(s, k, slot):
        vec = idx_v[s, 0, pl.ds(k * CH, CH)]
        tbv = (vec >> 3) * 8
        for u in range(CH):
            tb = pl.multiple_of(tbv[u], 8)
            pltpu.async_copy(
                table_hbm.at[pl.ds(tb, 8), :],
                rows_v.at[slot, pl.ds(u * 8, 8), :],
                gsems[slot],
            )

    def drain(slot):
        pltpu.make_async_copy(table_hbm.at[pl.ds(0, CH * 8), :],
                              rows_v.at[slot], gsems[slot]).wait()

    def transpose(s, k, slot, obs):
        rows = rows_v.at[slot]
        vec = idx_v[s, 0, pl.ds(k * CH, CH)]
        rowv = lax.iota(jnp.int32, L) * 8 + (vec & 7)

        def per_f(f, carry):
            colv = jnp.zeros((L,), jnp.int32) + f
            vals = plsc.load_gather(rows, [rowv, colv])
            obuf[obs, f, pl.ds(k * CH, CH)] = vals
            return carry

        lax.fori_loop(0, d, per_f, 0, unroll=8)

    def write(s, obs):
        pltpu.async_copy(obuf.at[obs], out_hbm.at[s, :, pl.ds(wid * BB, BB)],
                         wsems[obs])

    def wwait(obs):
        pltpu.make_async_copy(obuf.at[obs], out_hbm.at[0, :, pl.ds(0, BB)],
                              wsems[obs]).wait()

    issue(0, 0, 0)
    issue(0, 1, 1)
    issue(0, 2, 2)

    def body(i, carry):
        for obs in range(2):
            s = 2 * i + obs

            @pl.when(s >= 2)
            def _():
                wwait(obs)

            for k in range(UPS):
                slot = k % NBUF
                drain(slot)
                transpose(s, k, slot, obs)
                nxt = k + NBUF - 1
                if nxt < UPS:
                    issue(s, nxt, (nxt % NBUF))
                else:
                    @pl.when(s + 1 < seq)
                    def _():
                        issue(s + 1, nxt - UPS, nxt % NBUF)

            write(s, obs)
        return carry

    lax.fori_loop(0, seq // 2, body, 0, unroll=False)

    wwait(0)
    wwait(1)


def kernel(token_ids, weights):
    bsz, seq = token_ids.shape
    n, d = weights.shape
    assert bsz == NW * BB and d == 64 and seq % 2 == 0

    idx4 = (token_ids.astype(jnp.int32)
            .reshape(NW, BB, seq)
            .transpose(0, 2, 1)
            .reshape(NW, seq, 1, BB))

    emb = functools.partial(
        pl.kernel,
        mesh=plsc.VectorSubcoreMesh(core_axis_name="c", subcore_axis_name="s"),
        out_type=jax.ShapeDtypeStruct((seq, d, bsz), jnp.float32),
        scratch_types=[
            pltpu.VMEM((seq, 1, BB), jnp.int32),
            pltpu.VMEM((NBUF, CH * 8, d), jnp.float32),
            pltpu.VMEM((2, d, BB), jnp.float32),
            pltpu.SemaphoreType.DMA,
            pltpu.SemaphoreType.DMA,
            pltpu.SemaphoreType.DMA,
            pltpu.SemaphoreType.DMA,
            pltpu.SemaphoreType.DMA,
            pltpu.SemaphoreType.DMA,
        ],
        compiler_params=pltpu.CompilerParams(needs_layout_passes=False),
    )(functools.partial(_emb_body, seq, d))

    out_t = emb(weights, idx4)
    return out_t.transpose(2, 0, 1)
